# baseline (speedup 1.0000x reference)
import functools
import jax
import jax.numpy as jnp
from jax import lax
from jax.experimental import pallas as pl
from jax.experimental.pallas import tpu as pltpu
from jax.experimental.pallas import tpu_sc as plsc

B = 4096
L = 200
D = 64
CHUNKS = ((0, 104), (104, 96))
NBUF = 2
SLAB = 64
OGRP = 8
TBLK = 1024
VPAD = 200704


def _norm_kernel(w_ref, o_ref):
    w = w_ref[...]
    s = jnp.sum(w * w, axis=1, keepdims=True)
    n = jnp.maximum(jnp.sqrt(s), 1e-12)
    o_ref[:, pl.ds(0, D)] = w / n


def _normalize_table(w_r_table):
    return pl.pallas_call(
        _norm_kernel,
        grid=(VPAD // TBLK,),
        in_specs=[pl.BlockSpec((TBLK, D), lambda i: (i, 0))],
        out_specs=pl.BlockSpec((TBLK, 128), lambda i: (i, 0)),
        out_shape=jax.ShapeDtypeStruct((VPAD, 128), jnp.float32),
    )(w_r_table)


def _sc_kernel(rid_hbm, e_hbm, tnorm_hbm, out_hbm,
               idx_all, w_v, e_v, o_v, sems):
    info = plsc.get_sparse_core_info()
    nc = info.num_cores
    wid = lax.axis_index("s") * nc + lax.axis_index("c")
    b_per_w = B // (nc * info.num_subcores)
    base = wid * b_per_w

    def issue(bi, slot):
        for off, c in CHUNKS:
            pltpu.async_copy(
                tnorm_hbm.at[idx_all.at[pl.ds((bi % SLAB) * L + off, c)]],
                w_v.at[slot].at[pl.ds(off, c)], sems.at[slot])
        pltpu.async_copy(e_hbm.at[base + bi], e_v.at[slot], sems.at[slot])

    def drain(slot):
        for off, c in CHUNKS:
            pltpu.make_async_copy(tnorm_hbm.at[pl.ds(0, c)],
                                  w_v.at[slot].at[pl.ds(off, c)],
                                  sems.at[slot]).wait()
        pltpu.make_async_copy(e_hbm.at[0], e_v.at[slot],
                              sems.at[slot]).wait()

    rot = [(jnp.arange(16, dtype=jnp.int32) + sh) & 15 for sh in (8, 4, 2, 1)]
    dnums = lax.GatherDimensionNumbers(
        offset_dims=(), collapsed_slice_dims=(0,), start_index_map=(0,))

    def _allsum(v):
        for idx in rot:
            p = lax.gather(v, idx[:, None], dnums, (1,),
                           mode=lax.GatherScatterMode.PROMISE_IN_BOUNDS)
            v = v + p
        return v

    def compute(slot, bi):
        wb = w_v.at[slot]
        eb = e_v.at[slot]

        def l_body(l2, carry):
            a0, a1, a2, a3 = carry
            for u in range(2):
                l = l2 * 2 + u
                w0 = wb[l, pl.ds(0, 16)]
                w1 = wb[l, pl.ds(16, 16)]
                w2 = wb[l, pl.ds(32, 16)]
                w3 = wb[l, pl.ds(48, 16)]
                e0 = eb[l, pl.ds(0, 16)]
                e1 = eb[l, pl.ds(16, 16)]
                e2 = eb[l, pl.ds(32, 16)]
                e3 = eb[l, pl.ds(48, 16)]
                d = _allsum(e0 * w0 + e1 * w1 + e2 * w2 + e3 * w3)
                a0 = a0 + (e0 - d * w0)
                a1 = a1 + (e1 - d * w1)
                a2 = a2 + (e2 - d * w2)
                a3 = a3 + (e3 - d * w3)
            return (a0, a1, a2, a3)

        z = jnp.zeros((16,), jnp.float32)
        a0, a1, a2, a3 = lax.fori_loop(0, L // 2, l_body, (z, z, z, z))
        inv = jnp.float32(1.0 / L)
        ob = (bi % OGRP) * D
        o_v[pl.ds(ob + 0, 16)] = a0 * inv
        o_v[pl.ds(ob + 16, 16)] = a1 * inv
        o_v[pl.ds(ob + 32, 16)] = a2 * inv
        o_v[pl.ds(ob + 48, 16)] = a3 * inv

        @pl.when(bi % OGRP == OGRP - 1)
        def _():
            pltpu.sync_copy(
                o_v, out_hbm.at[pl.ds((base + bi - (OGRP - 1)) * D,
                                      OGRP * D)])

    def slab_body(g, _):
        g0 = g * SLAB
        pltpu.sync_copy(rid_hbm.at[pl.ds((base + g0) * L, SLAB * L)],
                        idx_all)
        issue(g0, 0)

        def pair_body(p, _):
            for s2 in range(NBUF):
                bi = g0 + p * NBUF + s2
                drain(s2)

                @pl.when(bi + 1 < g0 + SLAB)
                def _():
                    issue(bi + 1, (s2 + 1) % NBUF)

                compute(s2, bi)
            return 0

        lax.fori_loop(0, SLAB // NBUF, pair_body, 0)
        return 0

    lax.fori_loop(0, b_per_w // SLAB, slab_body, 0)


@jax.jit
def _run(rid_flat, batch_nei_e_emb, w_r_table):
    tnorm = _normalize_table(w_r_table)
    mesh = plsc.VectorSubcoreMesh(core_axis_name="c", subcore_axis_name="s")
    kfn = functools.partial(
        pl.kernel,
        mesh=mesh,
        out_type=jax.ShapeDtypeStruct((B * D,), jnp.float32),
        scratch_types=[
            pltpu.VMEM((SLAB * L,), jnp.int32),
            pltpu.VMEM((NBUF, L, 128), jnp.float32),
            pltpu.VMEM((NBUF, L, D), jnp.float32),
            pltpu.VMEM((OGRP * D,), jnp.float32),
            pltpu.SemaphoreType.DMA((NBUF,)),
        ],
    )(_sc_kernel)
    return kfn(rid_flat, batch_nei_e_emb, tnorm).reshape(B, D)


def kernel(batch_nei_rid, batch_nei_e_emb, w_r_table):
    return _run(batch_nei_rid.reshape(-1), batch_nei_e_emb, w_r_table)
